# baseline (speedup 1.0000x reference)
import jax
import jax.numpy as jnp
from jax import lax
from jax.experimental import pallas as pl
from jax.experimental.pallas import tpu as pltpu
from jax.experimental.pallas import tpu_sc as plsc

VOCAB = 1000
EMBED = 64
NUM_CLASS = 10
BATCH = 16384
HIST = 200

NPAIR = NUM_CLASS // 2
PROWS = 8
NC, NS, LANES = 2, 16, 16
NW = NC * NS
SPW = BATCH // NW
CHUNK = 128
NCHUNK = SPW // CHUNK
GRP = CHUNK // LANES


def _p_body(emb_ref, we_ref, wo_ref, be_ref, bo_ref, out_ref):
    pe = lax.dot_general(
        we_ref[...], emb_ref[...], (((1,), (1,)), ((), ())),
        preferred_element_type=jnp.float32,
    )
    po = lax.dot_general(
        wo_ref[...], emb_ref[...], (((1,), (1,)), ((), ())),
        preferred_element_type=jnp.float32,
    )
    pe = (pe + be_ref[...]) * (1.0 / HIST)
    po = (po + bo_ref[...]) * (1.0 / HIST)
    ue = lax.bitcast_convert_type(pe, jnp.uint32) + jnp.uint32(0x8000)
    uo = lax.bitcast_convert_type(po, jnp.uint32) + jnp.uint32(0x8000)
    word = (ue >> 16) | (uo & jnp.uint32(0xFFFF0000))
    out_ref[...] = lax.bitcast_convert_type(word, jnp.int32)


def _make_p(emb_table, w_even, w_odd, b_even, b_odd):
    return pl.pallas_call(
        _p_body,
        out_shape=jax.ShapeDtypeStruct((PROWS, VOCAB), jnp.int32),
    )(emb_table, w_even, w_odd, b_even, b_odd)


def _sc_body(p_hbm, text_hbm, out_hbm, p_v, text_v, out_v, psem, tsem0, tsem1):
    wid = lax.axis_index("s") * NC + lax.axis_index("c")
    base = wid * SPW
    lanes = lax.iota(jnp.int32, 16)
    cvecs = [jnp.full((16,), c, jnp.int32) for c in range(NUM_CLASS)]
    p_views = [p_v.at[pl.ds(pc * VOCAB, VOCAB)] for pc in range(NPAIR)]
    himask = jnp.full((16,), -65536, jnp.int32)
    UNROLL = 8

    CB = CHUNK * HIST
    tsems = [tsem0, tsem1]
    tviews = [text_v.at[pl.ds(b * CB, CB)] for b in range(2)]

    def chunk_copy(ck):
        return pltpu.async_copy(
            text_hbm.at[pl.ds((base + ck * CHUNK) * HIST, CB)],
            tviews[ck % 2],
            tsems[ck % 2],
        )

    pcopy = pltpu.async_copy(p_hbm.at[pl.ds(0, NPAIR * VOCAB)], p_v, psem)
    pending = chunk_copy(0)
    pcopy.wait()

    for ck in range(NCHUNK):
        pending.wait()
        if ck + 1 < NCHUNK:
            pending = chunk_copy(ck + 1)
        for g in range(GRP):
            tbase = (
                jnp.full((16,), (ck % 2) * CB + g * LANES * HIST, jnp.int32)
                + lanes * HIST
            )

            def jbody(jb, carry, tbase=tbase):
                accs = list(carry[:NUM_CLASS])
                idxs = carry[NUM_CLASS:]
                j1 = (jb + 1) * UNROLL
                nidxs = tuple(
                    plsc.load_gather(text_v, [tbase + (j1 + u)])
                    for u in range(UNROLL)
                )
                for u in range(UNROLL):
                    for pc in range(NPAIR):
                        w = plsc.load_gather(p_views[pc], [idxs[u]])
                        lo = plsc.bitcast(w << 16, jnp.float32)
                        hi = plsc.bitcast(w & himask, jnp.float32)
                        accs[2 * pc] = accs[2 * pc] + lo
                        accs[2 * pc + 1] = accs[2 * pc + 1] + hi
                return tuple(accs) + nidxs

            idxs0 = tuple(
                plsc.load_gather(text_v, [tbase + u]) for u in range(UNROLL)
            )
            carry = lax.fori_loop(
                0, HIST // UNROLL, jbody,
                tuple(jnp.zeros((16,), jnp.float32) for _ in range(NUM_CLASS))
                + idxs0,
            )
            accs = carry[:NUM_CLASS]

            m = accs[0]
            for c in range(1, NUM_CLASS):
                m = jnp.maximum(m, accs[c])
            es = [jnp.exp(a - m) for a in accs]
            tot = es[0]
            for c in range(1, NUM_CLASS):
                tot = tot + es[c]
            orows = jnp.full((16,), ck * CHUNK + g * LANES, jnp.int32) + lanes
            for c in range(NUM_CLASS):
                plsc.store_scatter(out_v, [orows, cvecs[c]], es[c] / tot)

    pltpu.sync_copy(out_v, out_hbm.at[pl.ds(base, SPW), :])


_sc_call = pl.kernel(
    _sc_body,
    out_type=jax.ShapeDtypeStruct((BATCH, NUM_CLASS), jnp.float32),
    mesh=plsc.VectorSubcoreMesh(core_axis_name="c", subcore_axis_name="s"),
    scratch_types=[
        pltpu.VMEM((NPAIR * VOCAB,), jnp.int32),
        pltpu.VMEM((2 * CHUNK * HIST + 16,), jnp.int32),
        pltpu.VMEM((SPW, NUM_CLASS), jnp.float32),
        pltpu.SemaphoreType.DMA,
        pltpu.SemaphoreType.DMA,
        pltpu.SemaphoreType.DMA,
    ],
    compiler_params=pltpu.CompilerParams(
        use_tc_tiling_on_sc=False, needs_layout_passes=False
    ),
)


def kernel(text, emb_table, fc_w, fc_b):
    text = text.astype(jnp.int32)
    w_even = jnp.zeros((PROWS, EMBED), jnp.float32).at[:NPAIR].set(fc_w[0::2])
    w_odd = jnp.zeros((PROWS, EMBED), jnp.float32).at[:NPAIR].set(fc_w[1::2])
    b_even = jnp.zeros((PROWS, 1), jnp.float32).at[:NPAIR, 0].set(fc_b[0::2])
    b_odd = jnp.zeros((PROWS, 1), jnp.float32).at[:NPAIR, 0].set(fc_b[1::2])
    p = _make_p(emb_table, w_even, w_odd, b_even, b_odd)
    return _sc_call(p.reshape(-1), text.reshape(-1))


# speedup vs baseline: 1.0184x; 1.0184x over previous
import jax
import jax.numpy as jnp
from jax import lax
from jax.experimental import pallas as pl
from jax.experimental.pallas import tpu as pltpu
from jax.experimental.pallas import tpu_sc as plsc

VOCAB = 1000
EMBED = 64
NUM_CLASS = 10
BATCH = 16384
HIST = 200

NPAIR = NUM_CLASS // 2
PROWS = 8
NC, NS, LANES = 2, 16, 16
NW = NC * NS
SPW = BATCH // NW
CHUNK = 128
NCHUNK = SPW // CHUNK
GRP = CHUNK // LANES


def _p_body(emb_ref, we_ref, wo_ref, be_ref, bo_ref, out_ref):
    pe = lax.dot_general(
        we_ref[...], emb_ref[...], (((1,), (1,)), ((), ())),
        preferred_element_type=jnp.float32,
    )
    po = lax.dot_general(
        wo_ref[...], emb_ref[...], (((1,), (1,)), ((), ())),
        preferred_element_type=jnp.float32,
    )
    pe = (pe + be_ref[...]) * (1.0 / HIST)
    po = (po + bo_ref[...]) * (1.0 / HIST)
    ue = lax.bitcast_convert_type(pe, jnp.uint32) + jnp.uint32(0x8000)
    uo = lax.bitcast_convert_type(po, jnp.uint32) + jnp.uint32(0x8000)
    word = (ue >> 16) | (uo & jnp.uint32(0xFFFF0000))
    out_ref[...] = lax.bitcast_convert_type(word, jnp.int32)


def _make_p(emb_table, w_even, w_odd, b_even, b_odd):
    return pl.pallas_call(
        _p_body,
        out_shape=jax.ShapeDtypeStruct((PROWS, VOCAB), jnp.int32),
    )(emb_table, w_even, w_odd, b_even, b_odd)


def _sc_body(p_hbm, text_hbm, out_hbm, p_v, text_v, out_v, psem, tsem0, tsem1):
    wid = lax.axis_index("s") * NC + lax.axis_index("c")
    base = wid * SPW
    lanes = lax.iota(jnp.int32, 16)
    cvecs = [jnp.full((16,), c, jnp.int32) for c in range(NUM_CLASS)]
    p_views = [p_v.at[pl.ds(pc * VOCAB, VOCAB)] for pc in range(NPAIR)]
    himask = jnp.full((16,), -65536, jnp.int32)
    UNROLL = 8

    CB = CHUNK * HIST
    tsems = [tsem0, tsem1]
    tviews = [text_v.at[pl.ds(b * CB, CB)] for b in range(2)]

    def chunk_copy(ck):
        return pltpu.async_copy(
            text_hbm.at[pl.ds((base + ck * CHUNK) * HIST, CB)],
            tviews[ck % 2],
            tsems[ck % 2],
        )

    pcopy = pltpu.async_copy(p_hbm.at[pl.ds(0, NPAIR * VOCAB)], p_v, psem)
    pending = chunk_copy(0)
    pcopy.wait()

    for ck in range(NCHUNK):
        pending.wait()
        if ck + 1 < NCHUNK:
            pending = chunk_copy(ck + 1)
        tview = tviews[ck % 2]
        for g in range(GRP):
            tbase = (jnp.full((16,), g * LANES, jnp.int32) + lanes) * HIST

            def jbody(jb, accs, tbase=tbase, tview=tview):
                j0 = jb * UNROLL
                idxs = [
                    plsc.load_gather(tview, [tbase + (j0 + u)])
                    for u in range(UNROLL)
                ]
                accs = list(accs)
                for u in range(UNROLL):
                    for pc in range(NPAIR):
                        w = plsc.load_gather(p_views[pc], [idxs[u]])
                        lo = plsc.bitcast(w << 16, jnp.float32)
                        hi = plsc.bitcast(w & himask, jnp.float32)
                        accs[2 * pc] = accs[2 * pc] + lo
                        accs[2 * pc + 1] = accs[2 * pc + 1] + hi
                return tuple(accs)

            accs = lax.fori_loop(
                0, HIST // UNROLL, jbody,
                tuple(jnp.zeros((16,), jnp.float32) for _ in range(NUM_CLASS)),
            )

            m = accs[0]
            for c in range(1, NUM_CLASS):
                m = jnp.maximum(m, accs[c])
            es = [jnp.exp(a - m) for a in accs]
            tot = es[0]
            for c in range(1, NUM_CLASS):
                tot = tot + es[c]
            orows = jnp.full((16,), ck * CHUNK + g * LANES, jnp.int32) + lanes
            for c in range(NUM_CLASS):
                plsc.store_scatter(out_v, [orows, cvecs[c]], es[c] / tot)

    pltpu.sync_copy(out_v, out_hbm.at[pl.ds(base, SPW), :])


_sc_call = pl.kernel(
    _sc_body,
    out_type=jax.ShapeDtypeStruct((BATCH, NUM_CLASS), jnp.float32),
    mesh=plsc.VectorSubcoreMesh(core_axis_name="c", subcore_axis_name="s"),
    scratch_types=[
        pltpu.VMEM((NPAIR * VOCAB,), jnp.int32),
        pltpu.VMEM((2 * CHUNK * HIST,), jnp.int32),
        pltpu.VMEM((SPW, NUM_CLASS), jnp.float32),
        pltpu.SemaphoreType.DMA,
        pltpu.SemaphoreType.DMA,
        pltpu.SemaphoreType.DMA,
    ],
    compiler_params=pltpu.CompilerParams(
        use_tc_tiling_on_sc=False, needs_layout_passes=False
    ),
)


def kernel(text, emb_table, fc_w, fc_b):
    text = text.astype(jnp.int32)
    w_even = jnp.zeros((PROWS, EMBED), jnp.float32).at[:NPAIR].set(fc_w[0::2])
    w_odd = jnp.zeros((PROWS, EMBED), jnp.float32).at[:NPAIR].set(fc_w[1::2])
    b_even = jnp.zeros((PROWS, 1), jnp.float32).at[:NPAIR, 0].set(fc_b[0::2])
    b_odd = jnp.zeros((PROWS, 1), jnp.float32).at[:NPAIR, 0].set(fc_b[1::2])
    p = _make_p(emb_table, w_even, w_odd, b_even, b_odd)
    return _sc_call(p.reshape(-1), text.reshape(-1))
